# baseline (speedup 1.0000x reference)
import functools

import jax
import jax.numpy as jnp
from jax import lax
from jax.experimental import pallas as pl
from jax.experimental.pallas import tpu as pltpu
from jax.experimental.pallas import tpu_sc as plsc

_BATCH = 4096
_HIST = 200
_HIDDEN = 64
_SPLITS = ((0, 96), (96, 104))


def _make_gather():
    info = plsc.get_sparse_core_info()
    nw = info.num_cores * info.num_subcores
    rows_per_w = _BATCH // nw
    mesh = plsc.VectorSubcoreMesh(core_axis_name="c", subcore_axis_name="s")

    @functools.partial(
        pl.kernel,
        mesh=mesh,
        out_type=jax.ShapeDtypeStruct((_BATCH, _HIST, _HIDDEN), jnp.float32),
        scratch_types=[
            pltpu.VMEM((rows_per_w, _HIST), jnp.int32),
            pltpu.VMEM((_HIST, _HIDDEN), jnp.float32),
            pltpu.VMEM((_HIST, _HIDDEN), jnp.float32),
            pltpu.SemaphoreType.DMA,
            pltpu.SemaphoreType.DMA,
            pltpu.SemaphoreType.DMA,
            pltpu.SemaphoreType.DMA,
        ],
        compiler_params=pltpu.CompilerParams(use_tc_tiling_on_sc=False),
    )
    def gather_kernel(idx_hbm, table_hbm, out_hbm, idx_v, rows0, rows1,
                      si0, si1, so0, so1):
        wid = lax.axis_index("s") * info.num_cores + lax.axis_index("c")
        base = wid * rows_per_w
        pltpu.sync_copy(idx_hbm.at[pl.ds(base, rows_per_w)], idx_v)

        def fire(r, rows, sem):
            for off, width in _SPLITS:
                pltpu.async_copy(
                    table_hbm.at[idx_v.at[r, pl.ds(off, width)]],
                    rows.at[pl.ds(off, width)],
                    sem,
                )

        def drain(r, rows, sem):
            for off, width in _SPLITS:
                pltpu.make_async_copy(
                    table_hbm.at[idx_v.at[r, pl.ds(off, width)]],
                    rows.at[pl.ds(off, width)],
                    sem,
                ).wait()

        def write(r, rows, sem):
            return pltpu.async_copy(rows, out_hbm.at[base + r], sem)

        def wait_write(r, rows, sem):
            pltpu.make_async_copy(rows, out_hbm.at[base + r], sem).wait()

        bufs = ((rows0, si0, so0), (rows1, si1, so1))

        def body(rr, carry):
            for b in range(2):
                rows, si, so = bufs[b]
                o_rows, o_si, o_so = bufs[1 - b]
                r = 2 * rr + b

                @pl.when(r >= 2)
                def _():
                    wait_write(r - 2, rows, so)

                fire(r, rows, si)

                @pl.when(r >= 1)
                def _():
                    drain(r - 1, o_rows, o_si)
                    write(r - 1, o_rows, o_so)

            return carry

        lax.fori_loop(0, rows_per_w // 2, body, 0)

        last = rows_per_w - 1
        rows, si, so = bufs[last % 2]
        o_rows, o_si, o_so = bufs[1 - last % 2]
        drain(last, rows, si)
        write(last, rows, so)
        wait_write(last - 1, o_rows, o_so)
        wait_write(last, rows, so)

    return gather_kernel


_gather = _make_gather()


def kernel(item_ids, table):
    return _gather(item_ids.astype(jnp.int32), table)


# speedup vs baseline: 1.4290x; 1.4290x over previous
import functools

import jax
import jax.numpy as jnp
from jax import lax
from jax.experimental import pallas as pl
from jax.experimental.pallas import tpu as pltpu
from jax.experimental.pallas import tpu_sc as plsc

_BATCH = 4096
_HIST = 200
_HIDDEN = 64
_ITEMS = 1000000
_SPLITS = ((0, 96), (96, 104))


def _make_gather():
    info = plsc.get_sparse_core_info()
    nw = info.num_cores * info.num_subcores
    rows_per_w = _BATCH // nw
    mesh = plsc.VectorSubcoreMesh(core_axis_name="c", subcore_axis_name="s")

    @functools.partial(
        pl.kernel,
        mesh=mesh,
        out_type=jax.ShapeDtypeStruct((_BATCH, _HIST, 2 * _HIDDEN), jnp.float32),
        scratch_types=[
            pltpu.VMEM((rows_per_w, _HIST), jnp.int32),
            pltpu.VMEM((_HIST, _HIDDEN), jnp.float32),
            pltpu.VMEM((_HIST, _HIDDEN), jnp.float32),
            pltpu.SemaphoreType.DMA,
            pltpu.SemaphoreType.DMA,
            pltpu.SemaphoreType.DMA,
            pltpu.SemaphoreType.DMA,
        ],
        compiler_params=pltpu.CompilerParams(use_tc_tiling_on_sc=False),
    )
    def gather_kernel(idx_hbm, table_hbm, out_hbm, idx_v, rows0, rows1,
                      si0, si1, so0, so1):
        wid = lax.axis_index("s") * info.num_cores + lax.axis_index("c")
        base = wid * rows_per_w
        pltpu.sync_copy(idx_hbm.at[pl.ds(base, rows_per_w)], idx_v)

        def fire(r, rows, sem):
            for off, width in _SPLITS:
                pltpu.async_copy(
                    table_hbm.at[idx_v.at[r, pl.ds(off, width)]],
                    rows.at[pl.ds(off, width)],
                    sem,
                )

        def drain(r, rows, sem):
            for off, width in _SPLITS:
                pltpu.make_async_copy(
                    table_hbm.at[idx_v.at[r, pl.ds(off, width)]],
                    rows.at[pl.ds(off, width)],
                    sem,
                ).wait()

        def write(r, rows, sem):
            return pltpu.async_copy(
                rows, out_hbm.at[base + r, :, pl.ds(0, _HIDDEN)], sem)

        def wait_write(r, rows, sem):
            pltpu.make_async_copy(
                rows, out_hbm.at[base + r, :, pl.ds(0, _HIDDEN)], sem).wait()

        bufs = ((rows0, si0, so0), (rows1, si1, so1))

        def body(rr, carry):
            for b in range(2):
                rows, si, so = bufs[b]
                o_rows, o_si, o_so = bufs[1 - b]
                r = 2 * rr + b

                @pl.when(r >= 2)
                def _():
                    wait_write(r - 2, rows, so)

                fire(r, rows, si)

                @pl.when(r >= 1)
                def _():
                    drain(r - 1, o_rows, o_si)
                    write(r - 1, o_rows, o_so)

            return carry

        lax.fori_loop(0, rows_per_w // 2, body, 0)

        last = rows_per_w - 1
        rows, si, so = bufs[last % 2]
        o_rows, o_si, o_so = bufs[1 - last % 2]
        drain(last, rows, si)
        write(last, rows, so)
        wait_write(last - 1, o_rows, o_so)
        wait_write(last, rows, so)

    return gather_kernel


_gather = _make_gather()


def kernel(item_ids, table):
    idx2 = item_ids.astype(jnp.int32) * 2
    tpad = jnp.pad(table, ((0, 0), (0, _HIDDEN)))
    t2 = tpad.reshape(2 * _ITEMS, _HIDDEN)
    outp = _gather(idx2, t2)
    return outp[..., :_HIDDEN]
